# baseline (speedup 1.0000x reference)
import functools

import jax
import jax.numpy as jnp
from jax import lax
from jax.experimental import pallas as pl
from jax.experimental.pallas import tpu as pltpu
from jax.experimental.pallas import tpu_sc as plsc

D = 128
CH_BIG = 128
CH_SM = 72
NRING = 4


@functools.cache
def _build(n_rows, doc_w, qry_w):
    info = plsc.get_sparse_core_info()
    nc, ns = info.num_cores, info.num_subcores
    nw = nc * ns
    rpw = n_rows // nw
    doc_rg = NRING
    qry_rg = 2 * NRING
    doc_ng = rpw // doc_rg
    qry_ng = rpw // qry_rg
    gpp = 2
    rps = gpp * doc_rg
    npiece = doc_ng // gpp
    mesh = plsc.VectorSubcoreMesh(core_axis_name="c", subcore_axis_name="s")

    @functools.partial(
        pl.kernel,
        out_type=(
            jax.ShapeDtypeStruct((n_rows, doc_w, D), jnp.float32),
            jax.ShapeDtypeStruct((n_rows, qry_w, D), jnp.float32),
        ),
        mesh=mesh,
        scratch_types=[
            pltpu.VMEM((2, rps, doc_w), jnp.int32),
            pltpu.VMEM((rpw, qry_w), jnp.int32),
            pltpu.VMEM((NRING, CH_BIG, D), jnp.float32),
            pltpu.VMEM((NRING, CH_SM, D), jnp.float32),
            pltpu.SemaphoreType.DMA((2 * NRING,)),
            pltpu.SemaphoreType.DMA((2 * NRING,)),
            pltpu.SemaphoreType.DMA,
        ],
    )
    def k(table, doc_idx, qry_idx, doc_out, qry_out, didx_v, qidx_v, big_v,
          sm_v, gsem, osem, ssem):
        wid = lax.axis_index("s") * nc + lax.axis_index("c")
        row0 = wid * rpw
        pltpu.sync_copy(qry_idx.at[pl.ds(row0, rpw)], qidx_v)

        doc_slots = []
        for i in range(doc_rg):
            doc_slots.append((i, 0, CH_BIG, big_v, i, i))
            doc_slots.append((i, CH_BIG, doc_w - CH_BIG, sm_v, i, NRING + i))
        qry_slots = []
        for i in range(qry_rg):
            buf = big_v if i < NRING else sm_v
            qry_slots.append((i, 0, qry_w, buf, i % NRING, i))

        def stage_doc(p, h):
            pltpu.async_copy(
                doc_idx.at[pl.ds(row0 + p * rps, rps)], didx_v.at[h], ssem)

        def wait_stage():
            pltpu.make_async_copy(
                doc_idx.at[pl.ds(row0, rps)], didx_v.at[0], ssem).wait()

        def doc_gather(g, slot):
            i, c, n, buf, bi, si = slot
            h = lax.rem(lax.div(g, gpp), 2)
            lr = lax.rem(g, gpp) * doc_rg + i
            pltpu.async_copy(
                table.at[didx_v.at[h, lr, pl.ds(c, n)]],
                buf.at[bi, pl.ds(0, n)], gsem.at[si])

        def gather(idx_v, g, rg, slot):
            i, c, n, buf, bi, si = slot
            pltpu.async_copy(
                table.at[idx_v.at[g * rg + i, pl.ds(c, n)]],
                buf.at[bi, pl.ds(0, n)], gsem.at[si])

        def wait_gather(slot):
            _, c, n, buf, bi, si = slot
            src = table.at[pl.ds(0, n)] if n % 8 == 0 else qry_out.at[0]
            pltpu.make_async_copy(
                src, buf.at[bi, pl.ds(0, n)], gsem.at[si]).wait()

        def put(out, base, g, rg, slot):
            i, c, n, buf, bi, si = slot
            pltpu.async_copy(
                buf.at[bi, pl.ds(0, n)],
                out.at[base + g * rg + i, pl.ds(c, n)], osem.at[si])

        def wait_put(out, slot):
            _, c, n, buf, bi, si = slot
            pltpu.make_async_copy(
                buf.at[bi, pl.ds(0, n)], out.at[0, pl.ds(c, n)],
                osem.at[si]).wait()

        def run(idx_v, out, base, rg, ng, slots):
            for slot in slots:
                gather(idx_v, 0, rg, slot)

            def body(g, carry):
                for slot in slots:
                    wait_gather(slot)
                    put(out, base, g, rg, slot)

                @pl.when(g + 1 < ng)
                def _():
                    for slot in slots:
                        wait_put(out, slot)
                        gather(idx_v, g + 1, rg, slot)

                @pl.when(g + 1 == ng)
                def _():
                    for slot in slots:
                        wait_put(out, slot)

                return carry

            lax.fori_loop(0, ng, body, 0)

        stage_doc(0, 0)
        wait_stage()
        for slot in doc_slots:
            doc_gather(0, slot)
        stage_doc(1, 1)

        def doc_body(g, carry):
            for slot in doc_slots:
                wait_gather(slot)
                put(doc_out, row0, g, doc_rg, slot)

            @pl.when(
                jnp.logical_and(lax.rem(g, gpp) == gpp - 1,
                                g + 1 < doc_ng))
            def _():
                wait_stage()

            @pl.when(g + 1 < doc_ng)
            def _():
                for slot in doc_slots:
                    wait_put(doc_out, slot)
                    doc_gather(g + 1, slot)

            @pl.when(g + 1 == doc_ng)
            def _():
                for slot in doc_slots:
                    wait_put(doc_out, slot)

            @pl.when(
                jnp.logical_and(lax.rem(g, gpp) == gpp - 1,
                                lax.div(g, gpp) + 2 < npiece))
            def _():
                p_next = lax.div(g, gpp) + 2
                stage_doc(p_next, lax.rem(p_next, 2))

            return carry

        lax.fori_loop(0, doc_ng, doc_body, 0)
        run(qidx_v, qry_out, row0, qry_rg, qry_ng, qry_slots)

    return k


def kernel(doc, qry, table):
    k = _build(doc.shape[0], doc.shape[1], qry.shape[1])
    return k(table, doc, qry)
